# baseline (speedup 1.0000x reference)
import functools

import jax
import jax.numpy as jnp
from jax import lax
from jax.experimental import pallas as pl
from jax.experimental.pallas import tpu as pltpu
from jax.experimental.pallas import tpu_sc as plsc

B, T, W, D, G = 8, 2048, 16, 128, 512
MAX_SPAN_LENGTH = 16
BIG = T * MAX_SPAN_LENGTH
NCHUNK = G // 16


def _s16(v):
    return lax.sort(v, dimension=0, is_stable=False)


def _bmerge(x):
    if len(x) == 1:
        return [_s16(x[0])]
    half = len(x) // 2
    lo = [jnp.minimum(a, b) for a, b in zip(x[:half], x[half:])]
    hi = [jnp.maximum(a, b) for a, b in zip(x[:half], x[half:])]
    return _bmerge(lo) + _bmerge(hi)


def _merge(a, b):
    c = [lax.rev(v, (0,)) for v in reversed(b)]
    lo = [jnp.minimum(x, y) for x, y in zip(a, c)]
    hi = [jnp.maximum(x, y) for x, y in zip(a, c)]
    return _bmerge(lo) + _bmerge(hi)


def _sort512(vecs):
    runs = [[_s16(v)] for v in vecs]
    while len(runs) > 1:
        runs = [_merge(runs[i], runs[i + 1]) for i in range(0, len(runs), 2)]
    return runs[0]


@functools.cache
def _make_sc_main():
    i32 = jnp.int32
    f32 = jnp.float32
    vec_i = jax.ShapeDtypeStruct((B, G), i32)
    vec_f = jax.ShapeDtypeStruct((B, G), f32)

    @functools.partial(
        pl.kernel,
        mesh=plsc.VectorSubcoreMesh(core_axis_name="c", subcore_axis_name="s"),
        compiler_params=pltpu.CompilerParams(needs_layout_passes=False),
        out_type=(
            jax.ShapeDtypeStruct((B * G, D), f32),
            vec_i, vec_i, vec_i, vec_i,
            vec_f, vec_f,
        ),
        scratch_types=[
            pltpu.VMEM((2, G), i32),
            pltpu.VMEM((16,), i32),
            pltpu.VMEM((G,), i32),
            pltpu.VMEM((G // 4, D), f32),
            pltpu.VMEM((G,), i32),
            pltpu.VMEM((G,), i32),
            pltpu.VMEM((G,), i32),
            pltpu.VMEM((G,), i32),
            pltpu.VMEM((G,), f32),
            pltpu.VMEM((G,), f32),
            pltpu.SemaphoreType.DMA,
            pltpu.SemaphoreType.DMA,
            pltpu.SemaphoreType.DMA,
        ],
    )
    def body(table, goldc, lens, fv_out, sidx_out, ri_out, sb_out, se_out,
             fb_out, fe_out, gold_v, lens_v, gidx_b, rows_v,
             sidx_b, ri_b, sb_b, se_b, fb_b, fe_b, sem, sem2, sem3):
        c = lax.axis_index("c")
        s = lax.axis_index("s")
        w = c * 16 + s
        b = w // 4
        q = w % 4

        h1 = pltpu.async_copy(goldc.at[b], gold_v, sem)
        h2 = pltpu.async_copy(lens, lens_v, sem)
        h1.wait()
        h2.wait()
        lane = lax.iota(i32, 16)
        lnv = jnp.sum(jnp.where(lane == b, lens_v[...], 0))

        vecs = []
        for k in range(NCHUNK):
            pos = lax.iota(i32, 16) + k * 16
            gb = gold_v[0, pl.ds(k * 16, 16)]
            ge = gold_v[1, pl.ds(k * 16, 16)]
            key = gb * MAX_SPAN_LENGTH + (ge - gb)
            m = jnp.where(pos < lnv, key, BIG)
            vecs.append(m * G + pos)

        svecs = _sort512(vecs)

        sidx_v = []
        for k, sv in enumerate(svecs):
            si = jnp.where((sv >> 9) < BIG, sv >> 9, 0)
            sidx_v.append(si)

        for j in range(8):
            val = jnp.where(q < 2,
                            jnp.where(q == 0, sidx_v[j], sidx_v[8 + j]),
                            jnp.where(q == 2, sidx_v[16 + j], sidx_v[24 + j]))
            gidx_b[pl.ds((q * 8 + j) * 16, 16)] = val + b * BIG

        rows = G // 4
        nst = 8
        step = rows // nst
        ghs = [pltpu.async_copy(
                   table.at[gidx_b.at[pl.ds(q * rows + j * step, step)]],
                   rows_v.at[pl.ds(j * step, step)], sem)
               for j in range(nst)]

        @pl.when(q == 0)
        def _():
            for k, (sv, si) in enumerate(zip(svecs, sidx_v)):
                sl = pl.ds(k * 16, 16)
                sbv = si >> 4
                sev = sbv + (si & (MAX_SPAN_LENGTH - 1))
                sidx_b[sl] = si
                ri_b[sl] = sv & (G - 1)
                sb_b[sl] = sbv
                se_b[sl] = sev
                fb_b[sl] = sbv.astype(f32)
                fe_b[sl] = sev.astype(f32)
            hs = [pltpu.async_copy(src, dst, sem2)
                  for src, dst in ((sidx_b, sidx_out.at[b]),
                                   (ri_b, ri_out.at[b]),
                                   (sb_b, sb_out.at[b]),
                                   (se_b, se_out.at[b]),
                                   (fb_b, fb_out.at[b]),
                                   (fe_b, fe_out.at[b]))]
            for h in hs:
                h.wait()

        whs = []
        for j in range(nst):
            ghs[j].wait()
            whs.append(pltpu.async_copy(
                rows_v.at[pl.ds(j * step, step)],
                fv_out.at[pl.ds(w * rows + j * step, step)], sem3))
        for h in whs:
            h.wait()

    return body


def _mask_body(lens_ref, sq_ref, tri_ref):
    ln = lens_ref[pl.program_id(0)]
    ii = lax.broadcasted_iota(jnp.int32, (G, G), 0)
    jj = lax.broadcasted_iota(jnp.int32, (G, G), 1)
    vm = (ii < ln) & (jj < ln)
    sq_ref[0] = jnp.where(vm, 1.0, 0.0)
    tri_ref[0] = jnp.where(vm & (jj <= ii), 1.0, 0.0)


def _mask_call(lengths):
    mask = jax.ShapeDtypeStruct((B, G, G), jnp.float32)
    mspec = pl.BlockSpec((1, G, G), lambda b: (b, 0, 0))
    return pl.pallas_call(
        _mask_body,
        grid=(B,),
        in_specs=[pl.BlockSpec(memory_space=pltpu.SMEM)],
        out_specs=[mspec, mspec],
        out_shape=[mask, mask],
    )(lengths)


def kernel(span_vecs, span_mask, span_begin, span_end,
           gold_span_tensors, gold_spans_lengths, sequence_lengths):
    table = span_vecs.reshape(B * T * W, D)
    gold_c = jnp.transpose(gold_span_tensors, (0, 2, 1))
    lens16 = jnp.pad(gold_spans_lengths, (0, 8))

    (fv, sidx, reindex, sb, se, fb, fe) = _make_sc_main()(
        table, gold_c, lens16)

    sq, tri = _mask_call(gold_spans_lengths)

    return (fv.reshape(B, G, D),
            jnp.zeros((B, G, 1), jnp.float32),
            fb.reshape(B, G, 1),
            fe.reshape(B, G, 1),
            sq,
            tri,
            sb,
            se,
            sidx,
            reindex)


# speedup vs baseline: 1.0161x; 1.0161x over previous
import functools

import jax
import jax.numpy as jnp
from jax import lax
from jax.experimental import pallas as pl
from jax.experimental.pallas import tpu as pltpu
from jax.experimental.pallas import tpu_sc as plsc

B, T, W, D, G = 8, 2048, 16, 128, 512
MAX_SPAN_LENGTH = 16
BIG = T * MAX_SPAN_LENGTH
NCHUNK = G // 16


def _s16(v):
    return lax.sort(v, dimension=0, is_stable=False)


def _bmerge(x):
    if len(x) == 1:
        return [_s16(x[0])]
    half = len(x) // 2
    lo = [jnp.minimum(a, b) for a, b in zip(x[:half], x[half:])]
    hi = [jnp.maximum(a, b) for a, b in zip(x[:half], x[half:])]
    return _bmerge(lo) + _bmerge(hi)


def _merge(a, b):
    c = [lax.rev(v, (0,)) for v in reversed(b)]
    lo = [jnp.minimum(x, y) for x, y in zip(a, c)]
    hi = [jnp.maximum(x, y) for x, y in zip(a, c)]
    return _bmerge(lo) + _bmerge(hi)


def _sort512(vecs):
    runs = [[_s16(v)] for v in vecs]
    while len(runs) > 1:
        runs = [_merge(runs[i], runs[i + 1]) for i in range(0, len(runs), 2)]
    return runs[0]


@functools.cache
def _make_sc_gather():
    i32 = jnp.int32
    f32 = jnp.float32

    @functools.partial(
        pl.kernel,
        mesh=plsc.VectorSubcoreMesh(core_axis_name="c", subcore_axis_name="s"),
        compiler_params=pltpu.CompilerParams(needs_layout_passes=False),
        out_type=jax.ShapeDtypeStruct((B * G, D), f32),
        scratch_types=[
            pltpu.VMEM((2, G), i32),
            pltpu.VMEM((16,), i32),
            pltpu.VMEM((G,), i32),
            pltpu.VMEM((G // 4, D), f32),
            pltpu.SemaphoreType.DMA,
            pltpu.SemaphoreType.DMA,
        ],
    )
    def body(table, goldc, lens, fv_out, gold_v, lens_v, gidx_b, rows_v,
             sem, sem3):
        c = lax.axis_index("c")
        s = lax.axis_index("s")
        w = c * 16 + s
        b = w // 4
        q = w % 4

        h1 = pltpu.async_copy(goldc.at[b], gold_v, sem)
        h2 = pltpu.async_copy(lens, lens_v, sem)
        h1.wait()
        h2.wait()
        lane = lax.iota(i32, 16)
        lnv = jnp.sum(jnp.where(lane == b, lens_v[...], 0))

        vecs = []
        for k in range(NCHUNK):
            pos = lax.iota(i32, 16) + k * 16
            gb = gold_v[0, pl.ds(k * 16, 16)]
            ge = gold_v[1, pl.ds(k * 16, 16)]
            key = gb * MAX_SPAN_LENGTH + (ge - gb)
            m = jnp.where(pos < lnv, key, BIG)
            vecs.append(m * G + pos)

        svecs = _sort512(vecs)
        sidx_v = [jnp.where((sv >> 9) < BIG, sv >> 9, 0) for sv in svecs]

        for j in range(8):
            val = jnp.where(q < 2,
                            jnp.where(q == 0, sidx_v[j], sidx_v[8 + j]),
                            jnp.where(q == 2, sidx_v[16 + j], sidx_v[24 + j]))
            gidx_b[pl.ds((q * 8 + j) * 16, 16)] = val + b * BIG

        rows = G // 4
        nst = 8
        step = rows // nst
        ghs = [pltpu.async_copy(
                   table.at[gidx_b.at[pl.ds(q * rows + j * step, step)]],
                   rows_v.at[pl.ds(j * step, step)], sem)
               for j in range(nst)]

        whs = []
        for j in range(nst):
            ghs[j].wait()
            whs.append(pltpu.async_copy(
                rows_v.at[pl.ds(j * step, step)],
                fv_out.at[pl.ds(w * rows + j * step, step)], sem3))
        for h in whs:
            h.wait()

    return body


def _tc_body(lens_ref, gold_r_ref, gold_c_ref,
             sorted_ref, reindex_ref, spanb_ref, spane_ref,
             fb_ref, fe_ref, sq_ref, tri_ref):
    ln = lens_ref[pl.program_id(0)]

    gb_r = gold_r_ref[0, :, 0:1]
    ge_r = gold_r_ref[0, :, 1:2]
    gb_c = gold_c_ref[0, 0:1, :]
    ge_c = gold_c_ref[0, 1:2, :]
    pos_r = lax.broadcasted_iota(jnp.int32, (G, 1), 0)
    pos_c = lax.broadcasted_iota(jnp.int32, (1, G), 1)

    key_r = gb_r * MAX_SPAN_LENGTH + (ge_r - gb_r)
    key_c = gb_c * MAX_SPAN_LENGTH + (ge_c - gb_c)
    masked_r = jnp.where(pos_r < ln, key_r, BIG)
    masked_c = jnp.where(pos_c < ln, key_c, BIG)
    ck_r = masked_r * G + pos_r
    ck_c = masked_c * G + pos_c

    lt = (ck_c < ck_r).astype(jnp.int32)
    rank_r = jnp.sum(lt, axis=1, keepdims=True)
    eq = rank_r == pos_c
    pos_m = jnp.broadcast_to(pos_r, (G, G))
    val_m = jnp.broadcast_to(masked_r, (G, G))
    reindex = jnp.sum(jnp.where(eq, pos_m, 0), axis=0, keepdims=True)
    sortedm = jnp.sum(jnp.where(eq, val_m, 0), axis=0, keepdims=True)

    sidx = jnp.where(sortedm < BIG, sortedm, 0)
    sb = sidx >> 4
    se = sb + (sidx & (MAX_SPAN_LENGTH - 1))

    sorted_ref[0] = sidx
    reindex_ref[0] = reindex
    spanb_ref[0] = sb
    spane_ref[0] = se
    fb_ref[0] = sb.astype(jnp.float32)
    fe_ref[0] = se.astype(jnp.float32)

    ii = lax.broadcasted_iota(jnp.int32, (G, G), 0)
    jj = lax.broadcasted_iota(jnp.int32, (G, G), 1)
    vm = (ii < ln) & (jj < ln)
    sq_ref[0] = jnp.where(vm, 1.0, 0.0)
    tri_ref[0] = jnp.where(vm & (jj <= ii), 1.0, 0.0)


def _tc_call(lengths, gold, gold_c):
    vec = jax.ShapeDtypeStruct((B, 1, G), jnp.int32)
    vecf = jax.ShapeDtypeStruct((B, 1, G), jnp.float32)
    mask = jax.ShapeDtypeStruct((B, G, G), jnp.float32)
    vspec = pl.BlockSpec((1, 1, G), lambda b: (b, 0, 0))
    mspec = pl.BlockSpec((1, G, G), lambda b: (b, 0, 0))
    return pl.pallas_call(
        _tc_body,
        grid=(B,),
        in_specs=[
            pl.BlockSpec(memory_space=pltpu.SMEM),
            pl.BlockSpec((1, G, 2), lambda b: (b, 0, 0)),
            pl.BlockSpec((1, 2, G), lambda b: (b, 0, 0)),
        ],
        out_specs=[vspec, vspec, vspec, vspec, vspec, vspec, mspec, mspec],
        out_shape=[vec, vec, vec, vec, vecf, vecf, mask, mask],
    )(lengths, gold, gold_c)


def kernel(span_vecs, span_mask, span_begin, span_end,
           gold_span_tensors, gold_spans_lengths, sequence_lengths):
    table = span_vecs.reshape(B * T * W, D)
    gold_c = jnp.transpose(gold_span_tensors, (0, 2, 1))
    lens16 = jnp.pad(gold_spans_lengths, (0, 8))

    fv = _make_sc_gather()(table, gold_c, lens16)

    (sidx, reindex, sb, se, fb, fe, sq, tri) = _tc_call(
        gold_spans_lengths, gold_span_tensors, gold_c)

    return (fv.reshape(B, G, D),
            jnp.zeros((B, G, 1), jnp.float32),
            fb.reshape(B, G, 1),
            fe.reshape(B, G, 1),
            sq,
            tri,
            sb.reshape(B, G),
            se.reshape(B, G),
            sidx.reshape(B, G),
            reindex.reshape(B, G))


# speedup vs baseline: 1.0396x; 1.0231x over previous
import functools

import jax
import jax.numpy as jnp
from jax import lax
from jax.experimental import pallas as pl
from jax.experimental.pallas import tpu as pltpu
from jax.experimental.pallas import tpu_sc as plsc

B, T, W, D, G = 8, 2048, 16, 128, 512
MAX_SPAN_LENGTH = 16
BIG = T * MAX_SPAN_LENGTH
NCHUNK = G // 16


def _s16(v):
    return lax.sort(v, dimension=0, is_stable=False)


def _bmerge(x):
    if len(x) == 1:
        return [_s16(x[0])]
    half = len(x) // 2
    lo = [jnp.minimum(a, b) for a, b in zip(x[:half], x[half:])]
    hi = [jnp.maximum(a, b) for a, b in zip(x[:half], x[half:])]
    return _bmerge(lo) + _bmerge(hi)


def _merge(a, b):
    c = [lax.rev(v, (0,)) for v in reversed(b)]
    lo = [jnp.minimum(x, y) for x, y in zip(a, c)]
    hi = [jnp.maximum(x, y) for x, y in zip(a, c)]
    return _bmerge(lo) + _bmerge(hi)


def _sort512(vecs):
    runs = [[_s16(v)] for v in vecs]
    while len(runs) > 1:
        runs = [_merge(runs[i], runs[i + 1]) for i in range(0, len(runs), 2)]
    return runs[0]


@functools.cache
def _make_sc_gather():
    i32 = jnp.int32
    f32 = jnp.float32

    @functools.partial(
        pl.kernel,
        mesh=plsc.VectorSubcoreMesh(core_axis_name="c", subcore_axis_name="s"),
        compiler_params=pltpu.CompilerParams(needs_layout_passes=False),
        out_type=jax.ShapeDtypeStruct((B * G, D), f32),
        scratch_types=[
            pltpu.VMEM((3, G), i32),
            pltpu.VMEM((G,), i32),
            pltpu.VMEM((G // 4, D), f32),
            pltpu.SemaphoreType.DMA,
            pltpu.SemaphoreType.DMA,
        ],
    )
    def body(table, goldc, fv_out, gold_v, gidx_b, rows_v, sem, sem3):
        c = lax.axis_index("c")
        s = lax.axis_index("s")
        w = c * 16 + s
        b = w // 4
        q = w % 4

        pltpu.async_copy(goldc.at[b], gold_v, sem).wait()
        lnv = gold_v[2, pl.ds(0, 16)]

        vecs = []
        for k in range(NCHUNK):
            pos = lax.iota(i32, 16) + k * 16
            gb = gold_v[0, pl.ds(k * 16, 16)]
            ge = gold_v[1, pl.ds(k * 16, 16)]
            key = gb * MAX_SPAN_LENGTH + (ge - gb)
            m = jnp.where(pos < lnv, key, BIG)
            vecs.append(m * G + pos)

        svecs = _sort512(vecs)
        sidx_v = [jnp.where((sv >> 9) < BIG, sv >> 9, 0) for sv in svecs]

        for j in range(8):
            val = jnp.where(q < 2,
                            jnp.where(q == 0, sidx_v[j], sidx_v[8 + j]),
                            jnp.where(q == 2, sidx_v[16 + j], sidx_v[24 + j]))
            gidx_b[pl.ds((q * 8 + j) * 16, 16)] = val + b * BIG

        rows = G // 4
        nst = 8
        step = rows // nst
        ghs = [pltpu.async_copy(
                   table.at[gidx_b.at[pl.ds(q * rows + j * step, step)]],
                   rows_v.at[pl.ds(j * step, step)], sem)
               for j in range(nst)]

        whs = []
        for j in range(nst):
            ghs[j].wait()
            whs.append(pltpu.async_copy(
                rows_v.at[pl.ds(j * step, step)],
                fv_out.at[pl.ds(w * rows + j * step, step)], sem3))
        for h in whs:
            h.wait()

    return body


def _tc_body(lens_ref, gold_r_ref, gold_c_ref,
             sorted_ref, reindex_ref, spanb_ref, spane_ref,
             fb_ref, fe_ref, sq_ref, tri_ref):
    ln = lens_ref[pl.program_id(0)]

    gb_r = gold_r_ref[0, :, 0:1]
    ge_r = gold_r_ref[0, :, 1:2]
    gb_c = gold_c_ref[0, 0:1, :]
    ge_c = gold_c_ref[0, 1:2, :]
    pos_r = lax.broadcasted_iota(jnp.int32, (G, 1), 0)
    pos_c = lax.broadcasted_iota(jnp.int32, (1, G), 1)

    key_r = gb_r * MAX_SPAN_LENGTH + (ge_r - gb_r)
    key_c = gb_c * MAX_SPAN_LENGTH + (ge_c - gb_c)
    masked_r = jnp.where(pos_r < ln, key_r, BIG)
    masked_c = jnp.where(pos_c < ln, key_c, BIG)
    ck_r = masked_r * G + pos_r
    ck_c = masked_c * G + pos_c

    lt = (ck_c < ck_r).astype(jnp.int32)
    rank_r = jnp.sum(lt, axis=1, keepdims=True)
    eq = rank_r == pos_c
    pos_m = jnp.broadcast_to(pos_r, (G, G))
    val_m = jnp.broadcast_to(masked_r, (G, G))
    reindex = jnp.sum(jnp.where(eq, pos_m, 0), axis=0, keepdims=True)
    sortedm = jnp.sum(jnp.where(eq, val_m, 0), axis=0, keepdims=True)

    sidx = jnp.where(sortedm < BIG, sortedm, 0)
    sb = sidx >> 4
    se = sb + (sidx & (MAX_SPAN_LENGTH - 1))

    sorted_ref[0] = sidx
    reindex_ref[0] = reindex
    spanb_ref[0] = sb
    spane_ref[0] = se
    fb_ref[0] = sb.astype(jnp.float32)
    fe_ref[0] = se.astype(jnp.float32)

    ii = lax.broadcasted_iota(jnp.int32, (G, G), 0)
    jj = lax.broadcasted_iota(jnp.int32, (G, G), 1)
    vm = (ii < ln) & (jj < ln)
    sq_ref[0] = jnp.where(vm, 1.0, 0.0)
    tri_ref[0] = jnp.where(vm & (jj <= ii), 1.0, 0.0)


def _tc_call(lengths, gold, gold_c):
    vec = jax.ShapeDtypeStruct((B, 1, G), jnp.int32)
    vecf = jax.ShapeDtypeStruct((B, 1, G), jnp.float32)
    mask = jax.ShapeDtypeStruct((B, G, G), jnp.float32)
    vspec = pl.BlockSpec((1, 1, G), lambda b: (b, 0, 0))
    mspec = pl.BlockSpec((1, G, G), lambda b: (b, 0, 0))
    return pl.pallas_call(
        _tc_body,
        grid=(B,),
        in_specs=[
            pl.BlockSpec(memory_space=pltpu.SMEM),
            pl.BlockSpec((1, G, 2), lambda b: (b, 0, 0)),
            pl.BlockSpec((1, 2, G), lambda b: (b, 0, 0)),
        ],
        out_specs=[vspec, vspec, vspec, vspec, vspec, vspec, mspec, mspec],
        out_shape=[vec, vec, vec, vec, vecf, vecf, mask, mask],
    )(lengths, gold, gold_c)


def kernel(span_vecs, span_mask, span_begin, span_end,
           gold_span_tensors, gold_spans_lengths, sequence_lengths):
    table = span_vecs.reshape(B * T * W, D)
    gold_c = jnp.transpose(gold_span_tensors, (0, 2, 1))
    lens_row = jnp.broadcast_to(gold_spans_lengths[:, None, None], (B, 1, G))
    gold_c3 = jnp.concatenate([gold_c, lens_row], axis=1)

    fv = _make_sc_gather()(table, gold_c3)

    (sidx, reindex, sb, se, fb, fe, sq, tri) = _tc_call(
        gold_spans_lengths, gold_span_tensors, gold_c)

    return (fv.reshape(B, G, D),
            jnp.zeros((B, G, 1), jnp.float32),
            fb.reshape(B, G, 1),
            fe.reshape(B, G, 1),
            sq,
            tri,
            sb.reshape(B, G),
            se.reshape(B, G),
            sidx.reshape(B, G),
            reindex.reshape(B, G))
